# baseline (speedup 1.0000x reference)
import functools

import jax
import jax.numpy as jnp
from jax import lax
from jax.experimental import pallas as pl
from jax.experimental.pallas import tpu as pltpu
from jax.experimental.pallas import tpu_sc as plsc

_VOCAB = 100000
_DIM = 32
_B = 4096
_L = 50

_NC = 2
_NS = 16
_NW = _NC * _NS
_BCOLS = _B // _NW
_GROUP = 16
_CHUNK = _GROUP * _L
_NSTREAM = _BCOLS // _GROUP
_NBUF = 2


def _make_sc_gather():
    mesh = plsc.VectorSubcoreMesh(core_axis_name="c", subcore_axis_name="s")

    @functools.partial(
        pl.kernel,
        mesh=mesh,
        out_type=jax.ShapeDtypeStruct((_L, _DIM, _B), jnp.float32),
        scratch_types=[
            pltpu.VMEM((_L, _BCOLS), jnp.int32),
            pltpu.VMEM((_NSTREAM * _CHUNK,), jnp.int32),
            pltpu.VMEM((_NBUF, _CHUNK, _DIM), jnp.float32),
            pltpu.VMEM((_NBUF, _L, _DIM, _GROUP), jnp.float32),
            pltpu.SemaphoreType.DMA((_NBUF,)),
            pltpu.SemaphoreType.DMA((_NBUF,)),
        ],
        compiler_params=pltpu.CompilerParams(
            use_tc_tiling_on_sc=False, needs_layout_passes=False),
    )
    def sc_gather(idx_hbm, table_hbm, out_hbm, idx_v, flat_v, bufs, tbufs,
                  gsem, osem):
        wid = lax.axis_index("s") * _NC + lax.axis_index("c")
        col0 = wid * _BCOLS
        pltpu.sync_copy(idx_hbm.at[:, pl.ds(col0, _BCOLS)], idx_v)

        def build(i, carry):
            j = i // _L
            l = i - j * _L
            v = idx_v[l, pl.ds(j * _GROUP, _GROUP)]
            flat_v[pl.ds(j * _CHUNK + l * _GROUP, _GROUP)] = v
            return carry

        lax.fori_loop(0, _NSTREAM * _L, build, 0)

        def gather(j, b):
            return pltpu.make_async_copy(
                table_hbm.at[flat_v.at[pl.ds(j * _CHUNK, _CHUNK)]],
                bufs.at[b], gsem.at[b])

        def writeback(j, b):
            return pltpu.make_async_copy(
                tbufs.at[b],
                out_hbm.at[:, :, pl.ds(col0 + j * _GROUP, _GROUP)],
                osem.at[b])

        lane = lax.iota(jnp.int32, 16)
        dcols = [jnp.full((16,), d, jnp.int32) for d in range(_DIM)]

        def transpose(b):
            def trow(l, carry):
                rows = l * _GROUP + lane
                for d in range(_DIM):
                    v = plsc.load_gather(bufs.at[b], [rows, dcols[d]])
                    tbufs[b, l, d, :] = v
                return carry

            lax.fori_loop(0, _L, trow, 0)

        for j in range(_NBUF):
            gather(j, j).start()

        def body(j, carry):
            b = lax.rem(j, _NBUF)
            gather(j, b).wait()

            @pl.when(j >= _NBUF)
            def _():
                writeback(j - _NBUF, b).wait()

            transpose(b)
            writeback(j, b).start()

            @pl.when(j + _NBUF < _NSTREAM)
            def _():
                gather(j + _NBUF, b).start()

            return carry

        lax.fori_loop(0, _NSTREAM, body, 0)

        for j in range(_NSTREAM - _NBUF, _NSTREAM):
            writeback(j, j % _NBUF).wait()

    return sc_gather


_sc_gather = _make_sc_gather()


def kernel(words_seq, table):
    out_t = _sc_gather(words_seq.T, table)
    return jnp.transpose(out_t, (2, 0, 1))


# speedup vs baseline: 1.3562x; 1.3562x over previous
import functools

import jax
import jax.numpy as jnp
from jax import lax
from jax.experimental import pallas as pl
from jax.experimental.pallas import tpu as pltpu
from jax.experimental.pallas import tpu_sc as plsc

_VOCAB = 100000
_DIM = 32
_B = 4096
_L = 50

_NC = 2
_NS = 16
_NW = _NC * _NS
_BCOLS = _B // _NW
_GROUP = 16
_CHUNK = _GROUP * _L
_NSTREAM = _BCOLS // _GROUP
_NBUF = 2


def _make_sc_gather():
    mesh = plsc.VectorSubcoreMesh(core_axis_name="c", subcore_axis_name="s")

    @functools.partial(
        pl.kernel,
        mesh=mesh,
        out_type=jax.ShapeDtypeStruct((_L, _DIM, _B), jnp.float32),
        scratch_types=[
            pltpu.VMEM((_L, _BCOLS), jnp.int32),
            pltpu.VMEM((_NSTREAM * _CHUNK,), jnp.int32),
            pltpu.VMEM((_NBUF, _CHUNK, _DIM), jnp.float32),
            pltpu.VMEM((1, _L, _DIM, _GROUP + 8), jnp.float32),
            pltpu.SemaphoreType.DMA((_NBUF,)),
            pltpu.SemaphoreType.DMA((_NBUF,)),
        ],
        compiler_params=pltpu.CompilerParams(
            use_tc_tiling_on_sc=False, needs_layout_passes=False),
    )
    def sc_gather(idx_hbm, table_hbm, out_hbm, idx_v, flat_v, bufs, tbufs,
                  gsem, osem):
        wid = lax.axis_index("s") * _NC + lax.axis_index("c")
        col0 = wid * _BCOLS
        pltpu.sync_copy(idx_hbm.at[:, pl.ds(col0, _BCOLS)], idx_v)

        def build(i, carry):
            j = i // _L
            l = i - j * _L
            v = idx_v[l, pl.ds(j * _GROUP, _GROUP)]
            flat_v[pl.ds(j * _CHUNK + l * _GROUP, _GROUP)] = v
            return carry

        lax.fori_loop(0, _NSTREAM * _L, build, 0)

        def gather(j, b):
            return pltpu.make_async_copy(
                table_hbm.at[flat_v.at[pl.ds(j * _CHUNK, _CHUNK)]],
                bufs.at[b], gsem.at[b])

        def writeback(j, b):
            return pltpu.make_async_copy(
                tbufs.at[0, :, :, pl.ds(0, _GROUP)],
                out_hbm.at[:, :, pl.ds(col0 + j * _GROUP, _GROUP)],
                osem.at[b])

        lane = lax.iota(jnp.int32, 16)
        gvs = [jnp.full((16,), g, jnp.int32) for g in range(_GROUP)]

        def transpose(b):
            def trow(l, carry):
                lv = jnp.full((16,), 0, jnp.int32) + l
                for g in range(_GROUP):
                    for d0 in (0, 16):
                        v = bufs[b, l * _GROUP + g, pl.ds(d0, 16)]
                        plsc.store_scatter(
                            tbufs.at[0], [lv, d0 + lane, gvs[g]], v)
                return carry

            lax.fori_loop(0, _L, trow, 0)

        for j in range(_NBUF):
            gather(j, j).start()

        def body(j, carry):
            b = lax.rem(j, _NBUF)
            gather(j, b).wait()

            @pl.when(j >= 1)
            def _():
                writeback(j - 1, lax.rem(j - 1, _NBUF)).wait()

            transpose(b)
            writeback(j, b).start()

            @pl.when(j + _NBUF < _NSTREAM)
            def _():
                gather(j + _NBUF, b).start()

            return carry

        lax.fori_loop(0, _NSTREAM, body, 0)

        writeback(_NSTREAM - 1, (_NSTREAM - 1) % _NBUF).wait()

    return sc_gather


_sc_gather = _make_sc_gather()


def kernel(words_seq, table):
    out_t = _sc_gather(words_seq.T, table)
    return jnp.transpose(out_t, (2, 0, 1))


# speedup vs baseline: 1.5541x; 1.1460x over previous
import functools

import jax
import jax.numpy as jnp
from jax import lax
from jax.experimental import pallas as pl
from jax.experimental.pallas import tpu as pltpu
from jax.experimental.pallas import tpu_sc as plsc

_VOCAB = 100000
_DIM = 32
_B = 4096
_L = 50

_NC = 2
_NS = 16
_NW = _NC * _NS
_BCOLS = _B // _NW
_GROUP = 16
_CHUNK = _GROUP * _L
_NSTREAM = _BCOLS // _GROUP
_NBUF = 2


def _make_sc_gather():
    mesh = plsc.VectorSubcoreMesh(core_axis_name="c", subcore_axis_name="s")

    @functools.partial(
        pl.kernel,
        mesh=mesh,
        out_type=jax.ShapeDtypeStruct((_L, _DIM, _B), jnp.float32),
        scratch_types=[
            pltpu.VMEM((_L, _BCOLS), jnp.int32),
            pltpu.VMEM((_NSTREAM * _CHUNK,), jnp.int32),
            pltpu.VMEM((_NBUF, _CHUNK, _DIM), jnp.float32),
            pltpu.VMEM((_NBUF, _L, _DIM, _GROUP), jnp.float32),
            pltpu.SemaphoreType.DMA((_NBUF,)),
            pltpu.SemaphoreType.DMA((_NBUF,)),
        ],
        compiler_params=pltpu.CompilerParams(
            use_tc_tiling_on_sc=False, needs_layout_passes=False),
    )
    def sc_gather(idx_hbm, table_hbm, out_hbm, idx_v, flat_v, bufs, tbufs,
                  gsem, osem):
        wid = lax.axis_index("s") * _NC + lax.axis_index("c")
        col0 = wid * _BCOLS
        pltpu.sync_copy(idx_hbm.at[:, pl.ds(col0, _BCOLS)], idx_v)

        def build(i, carry):
            j = i // _L
            l = i - j * _L
            v = idx_v[l, pl.ds(j * _GROUP, _GROUP)]
            flat_v[pl.ds(j * _CHUNK + l * _GROUP, _GROUP)] = v
            return carry

        lax.fori_loop(0, _NSTREAM * _L, build, 0)

        def gather(j, b):
            return pltpu.make_async_copy(
                table_hbm.at[flat_v.at[pl.ds(j * _CHUNK, _CHUNK)]],
                bufs.at[b], gsem.at[b])

        def writeback(j, b):
            return pltpu.make_async_copy(
                tbufs.at[b],
                out_hbm.at[:, :, pl.ds(col0 + j * _GROUP, _GROUP)],
                osem.at[b])

        lane = lax.iota(jnp.int32, 16)

        def transpose(b):
            def trow(l, carry):
                lv = jnp.full((16,), 0, jnp.int32) + l
                for g in range(_GROUP):
                    gv = jnp.full((16,), g, jnp.int32)
                    for d0 in (0, 16):
                        v = bufs[b, l * _GROUP + g, pl.ds(d0, 16)]
                        plsc.store_scatter(
                            tbufs.at[b], [lv, d0 + lane, gv], v)
                return carry

            lax.fori_loop(0, _L, trow, 0)

        for j in range(_NBUF):
            gather(j, j).start()

        def body(j, carry):
            b = lax.rem(j, _NBUF)
            gather(j, b).wait()

            @pl.when(j >= _NBUF)
            def _():
                writeback(j - _NBUF, b).wait()

            transpose(b)
            writeback(j, b).start()

            @pl.when(j + _NBUF < _NSTREAM)
            def _():
                gather(j + _NBUF, b).start()

            return carry

        lax.fori_loop(0, _NSTREAM, body, 0)

        for j in range(_NSTREAM - _NBUF, _NSTREAM):
            writeback(j, j % _NBUF).wait()

    return sc_gather


_sc_gather = _make_sc_gather()


def kernel(words_seq, table):
    out_t = _sc_gather(words_seq.T, table)
    return jnp.transpose(out_t, (2, 0, 1))
